# baseline (speedup 1.0000x reference)
import jax
import jax.numpy as jnp
from jax.experimental import pallas as pl
from jax.experimental.pallas import tpu as pltpu

_CHUNK = 512
_NBUF = 8


def _copy(flags_hbm, buf, sems, block, slot):
    return pltpu.make_async_copy(
        flags_hbm.at[pl.ds(block * _CHUNK, _CHUNK), :],
        buf.at[slot],
        sems.at[slot],
    )


def _fbe_block(flags_hbm, emba_ref, out_ref, buf, sems):
    i = pl.program_id(0)
    nsteps = pl.num_programs(0)

    @pl.when(i == 0)
    def _prologue():
        for b in range(_NBUF):
            _copy(flags_hbm, buf, sems, b, b).start()

    slot = jax.lax.rem(i, _NBUF)
    _copy(flags_hbm, buf, sems, i, slot).wait()

    mask = (buf[slot] > 0.5).astype(jnp.float32)
    acc = jnp.dot(mask, emba_ref[:],
                  preferred_element_type=jnp.float32)
    d = out_ref.shape[1]
    sums = acc[:, :d]
    counts = acc[:, d:d + 1]
    out_ref[:] = sums / jnp.maximum(counts, 1.0)

    @pl.when(i + _NBUF < nsteps)
    def _refill():
        _copy(flags_hbm, buf, sems, i + _NBUF, slot).start()


def kernel(flags_matrix, emb):
    t, k = flags_matrix.shape
    k2, d = emb.shape
    emb_aug = jnp.concatenate([emb, jnp.ones((k2, 1), jnp.float32)], axis=1)
    grid = t // _CHUNK
    return pl.pallas_call(
        _fbe_block,
        grid=(grid,),
        in_specs=[
            pl.BlockSpec(memory_space=pl.ANY),
            pl.BlockSpec((k2, d + 1), lambda i: (0, 0)),
        ],
        out_specs=pl.BlockSpec((_CHUNK, d), lambda i: (i, 0)),
        out_shape=jax.ShapeDtypeStruct((t, d), jnp.float32),
        scratch_shapes=[
            pltpu.VMEM((_NBUF, _CHUNK, k), jnp.float32),
            pltpu.SemaphoreType.DMA((_NBUF,)),
        ],
        compiler_params=pltpu.CompilerParams(
            dimension_semantics=("arbitrary",),
        ),
    )(flags_matrix, emb_aug)


# speedup vs baseline: 1.0046x; 1.0046x over previous
import jax
import jax.numpy as jnp
from jax.experimental import pallas as pl
from jax.experimental.pallas import tpu as pltpu

_BT = 2048


def _fbe_block(flags_ref, emba_ref, out_ref):
    mask = (flags_ref[:] > 0.5).astype(jnp.float32)
    acc = jnp.dot(mask, emba_ref[:],
                  preferred_element_type=jnp.float32)
    d = out_ref.shape[1]
    sums = acc[:, :d]
    counts = acc[:, d:d + 1]
    out_ref[:] = sums / jnp.maximum(counts, 1.0)


def kernel(flags_matrix, emb):
    t, k = flags_matrix.shape
    k2, d = emb.shape
    emb_aug = jnp.concatenate([emb, jnp.ones((k2, 1), jnp.float32)], axis=1)
    grid = t // _BT
    return pl.pallas_call(
        _fbe_block,
        grid=(grid,),
        in_specs=[
            pl.BlockSpec((_BT, k), lambda i: (i, 0)),
            pl.BlockSpec((k2, d + 1), lambda i: (0, 0)),
        ],
        out_specs=pl.BlockSpec((_BT, d), lambda i: (i, 0)),
        out_shape=jax.ShapeDtypeStruct((t, d), jnp.float32),
        compiler_params=pltpu.CompilerParams(
            dimension_semantics=("arbitrary",),
        ),
    )(flags_matrix, emb_aug)
